# baseline (speedup 1.0000x reference)
import jax
import jax.numpy as jnp
from jax.experimental import pallas as pl
from jax.experimental.pallas import tpu as pltpu

_N_SPLIT = 8


def _copy_body(emb_ref, out_ref, sem):
    rows = emb_ref.shape[0]
    stripe = rows // _N_SPLIT
    for i in range(_N_SPLIT):
        pltpu.make_async_copy(
            emb_ref.at[pl.ds(i * stripe, stripe)],
            out_ref.at[pl.ds(i * stripe, stripe)],
            sem.at[i],
        ).start()
    for i in range(_N_SPLIT):
        pltpu.make_async_copy(
            emb_ref.at[pl.ds(i * stripe, stripe)],
            out_ref.at[pl.ds(i * stripe, stripe)],
            sem.at[i],
        ).wait()


def kernel(x, emb):
    seq_len = x.shape[1]
    return pl.pallas_call(
        _copy_body,
        out_shape=jax.ShapeDtypeStruct((seq_len, emb.shape[1]), emb.dtype),
        in_specs=[pl.BlockSpec(memory_space=pltpu.MemorySpace.HBM)],
        out_specs=pl.BlockSpec(memory_space=pltpu.MemorySpace.HBM),
        scratch_shapes=[pltpu.SemaphoreType.DMA((_N_SPLIT,))],
    )(emb[:seq_len])


# speedup vs baseline: 41.6683x; 41.6683x over previous
import jax
import jax.numpy as jnp
from jax.experimental import pallas as pl
from jax.experimental.pallas import tpu as pltpu

_BLOCK_ROWS = 512


def _copy_body(emb_ref, out_ref):
    out_ref[...] = emb_ref[...]


def kernel(x, emb):
    seq_len = x.shape[1]
    dim = emb.shape[1]
    grid = (seq_len // _BLOCK_ROWS,)
    return pl.pallas_call(
        _copy_body,
        out_shape=jax.ShapeDtypeStruct((seq_len, dim), emb.dtype),
        grid=grid,
        in_specs=[pl.BlockSpec((_BLOCK_ROWS, dim), lambda i: (i, 0))],
        out_specs=pl.BlockSpec((_BLOCK_ROWS, dim), lambda i: (i, 0)),
    )(emb[:seq_len])
